# baseline (speedup 1.0000x reference)
import functools

import jax
import jax.numpy as jnp
from jax import lax
from jax.experimental import pallas as pl
from jax.experimental.pallas import tpu as pltpu
from jax.experimental.pallas import tpu_sc as plsc

SEQ_LEN = 8192
D_MODEL = 1024
LANES = 16
NUM_WORKERS = 32
ROWS_PER_WORKER = SEQ_LEN // NUM_WORKERS
CHUNK_ROWS = 32
NUM_CHUNKS = ROWS_PER_WORKER // CHUNK_ROWS

_mesh = plsc.VectorSubcoreMesh(core_axis_name="c", subcore_axis_name="s")


@functools.partial(
    pl.kernel,
    mesh=_mesh,
    out_type=jax.ShapeDtypeStruct((SEQ_LEN, D_MODEL), jnp.float32),
    scratch_types=[
        pltpu.VMEM((CHUNK_ROWS, D_MODEL), jnp.float32),
        pltpu.VMEM((CHUNK_ROWS, D_MODEL), jnp.float32),
        pltpu.SemaphoreType.DMA,
        pltpu.SemaphoreType.DMA,
    ],
)
def _sc_add(x_hbm, emb_hbm, out_hbm, xbuf, ebuf, sem_x, sem_e):
    wid = lax.axis_index("s") * 2 + lax.axis_index("c")
    base = wid * ROWS_PER_WORKER

    def chunk_body(ci, carry):
        row0 = base + ci * CHUNK_ROWS
        cx = pltpu.async_copy(x_hbm.at[pl.ds(row0, CHUNK_ROWS), :], xbuf, sem_x)
        ce = pltpu.async_copy(emb_hbm.at[pl.ds(row0, CHUNK_ROWS), :], ebuf, sem_e)
        cx.wait()
        ce.wait()

        def row_body(r, rcarry):
            def vec_body(j, vcarry):
                sl = pl.ds(j * LANES, LANES)
                xbuf[r, sl] = xbuf[r, sl] + ebuf[r, sl]
                return vcarry

            return lax.fori_loop(0, D_MODEL // LANES, vec_body, rcarry)

        lax.fori_loop(0, CHUNK_ROWS, row_body, 0)
        pltpu.sync_copy(xbuf, out_hbm.at[pl.ds(row0, CHUNK_ROWS), :])
        return carry

    lax.fori_loop(0, NUM_CHUNKS, chunk_body, 0)


def kernel(x, emb):
    return _sc_add(x, emb[:SEQ_LEN])


# speedup vs baseline: 2.2044x; 2.2044x over previous
import functools

import jax
import jax.numpy as jnp
from jax import lax
from jax.experimental import pallas as pl
from jax.experimental.pallas import tpu as pltpu
from jax.experimental.pallas import tpu_sc as plsc

SEQ_LEN = 8192
D_MODEL = 1024
LANES = 16
NUM_WORKERS = 32
ROWS_PER_WORKER = SEQ_LEN // NUM_WORKERS
CHUNK_ROWS = 16
NUM_CHUNKS = ROWS_PER_WORKER // CHUNK_ROWS

_mesh = plsc.VectorSubcoreMesh(core_axis_name="c", subcore_axis_name="s")

_CHUNK = (CHUNK_ROWS, D_MODEL)


@functools.partial(
    pl.kernel,
    mesh=_mesh,
    out_type=jax.ShapeDtypeStruct((SEQ_LEN, D_MODEL), jnp.float32),
    scratch_types=[
        pltpu.VMEM(_CHUNK, jnp.float32),
        pltpu.VMEM(_CHUNK, jnp.float32),
        pltpu.VMEM(_CHUNK, jnp.float32),
        pltpu.VMEM(_CHUNK, jnp.float32),
        pltpu.VMEM(_CHUNK, jnp.float32),
        pltpu.VMEM(_CHUNK, jnp.float32),
        pltpu.SemaphoreType.DMA,
        pltpu.SemaphoreType.DMA,
        pltpu.SemaphoreType.DMA,
        pltpu.SemaphoreType.DMA,
        pltpu.SemaphoreType.DMA,
        pltpu.SemaphoreType.DMA,
    ],
)
def _sc_add(x_hbm, emb_hbm, out_hbm,
            xb0, xb1, eb0, eb1, ob0, ob1,
            sx0, sx1, se0, se1, so0, so1):
    xbufs = (xb0, xb1)
    ebufs = (eb0, eb1)
    obufs = (ob0, ob1)
    sxs = (sx0, sx1)
    ses = (se0, se1)
    sos = (so0, so1)

    wid = lax.axis_index("s") * 2 + lax.axis_index("c")
    base = wid * ROWS_PER_WORKER

    def rows_at(ci):
        return pl.ds(base + ci * CHUNK_ROWS, CHUNK_ROWS)

    def start_gather(ci, b):
        pltpu.async_copy(x_hbm.at[rows_at(ci), :], xbufs[b], sxs[b])
        pltpu.async_copy(emb_hbm.at[rows_at(ci), :], ebufs[b], ses[b])

    def wait_gather(b):
        pltpu.make_async_copy(x_hbm.at[rows_at(0), :], xbufs[b], sxs[b]).wait()
        pltpu.make_async_copy(emb_hbm.at[rows_at(0), :], ebufs[b], ses[b]).wait()

    def wait_scatter(b):
        pltpu.make_async_copy(obufs[b], out_hbm.at[rows_at(0), :], sos[b]).wait()

    start_gather(0, 0)

    def outer(g, carry):
        for b in (0, 1):
            ci = 2 * g + b
            @pl.when(ci + 1 < NUM_CHUNKS)
            def _():
                start_gather(ci + 1, 1 - b)

            wait_gather(b)

            @pl.when(ci >= 2)
            def _():
                wait_scatter(b)

            xbuf, ebuf, obuf = xbufs[b], ebufs[b], obufs[b]

            def row_body(r, rcarry):
                for j in range(D_MODEL // LANES):
                    sl = pl.ds(j * LANES, LANES)
                    obuf[r, sl] = xbuf[r, sl] + ebuf[r, sl]
                return rcarry

            lax.fori_loop(0, CHUNK_ROWS, row_body, 0)
            pltpu.async_copy(obuf, out_hbm.at[rows_at(ci), :], sos[b])
        return carry

    lax.fori_loop(0, NUM_CHUNKS // 2, outer, 0)
    wait_scatter(0)
    wait_scatter(1)


def kernel(x, emb):
    return _sc_add(x, emb[:SEQ_LEN])
